# initial kernel scaffold (unmeasured)
import jax
import jax.numpy as jnp
from jax import lax
from jax.experimental import pallas as pl
from jax.experimental.pallas import tpu as pltpu

N_DEV = 4
N_TILES = 4


def kernel(x, w_mat, scale_x, scale_w):
    m_global, k_shard = x.shape
    _, n = w_mat.shape
    m_per = m_global // N_DEV
    nt = n // N_TILES

    def body(x_ref, w_ref, sx_ref, sw_ref, out_ref,
             w_bf, send_buf, recv_buf, send_sems, recv_sems):
        t = pl.program_id(0)
        d = lax.axis_index("i")
        left = lax.rem(d + N_DEV - 1, N_DEV)
        right = lax.rem(d + 1, N_DEV)

        @pl.when(t == 0)
        def _():
            barrier_sem = pltpu.get_barrier_semaphore()
            pl.semaphore_signal(barrier_sem, inc=1, device_id=(left,),
                                device_id_type=pl.DeviceIdType.MESH)
            pl.semaphore_signal(barrier_sem, inc=1, device_id=(right,),
                                device_id_type=pl.DeviceIdType.MESH)
            pl.semaphore_wait(barrier_sem, 2)

        w_bf[...] = w_ref[...].astype(jnp.bfloat16)

        def partial(c):
            xc = x_ref[pl.ds(c * m_per, m_per), :].astype(jnp.bfloat16)
            return jnp.dot(xc, w_bf[...], preferred_element_type=jnp.float32)

        for s in range(N_DEV - 1):
            c = lax.rem(d + 2 * N_DEV - 1 - s, N_DEV)
            acc = partial(c)
            if s > 0:
                acc = acc + recv_buf[s - 1].astype(jnp.float32)
            send_buf[...] = acc.astype(jnp.bfloat16)
            rdma = pltpu.make_async_remote_copy(
                src_ref=send_buf,
                dst_ref=recv_buf.at[s],
                send_sem=send_sems.at[s],
                recv_sem=recv_sems.at[s],
                device_id=(right,),
                device_id_type=pl.DeviceIdType.MESH,
            )
            rdma.start()
            rdma.wait()

        acc = partial(d) + recv_buf[N_DEV - 2].astype(jnp.float32)
        scale = sx_ref[0, 0] * sw_ref[0, 0]
        out_ref[...] = jnp.maximum(acc * scale, 0.0)

    return pl.pallas_call(
        body,
        grid=(N_TILES,),
        in_specs=[
            pl.BlockSpec((m_global, k_shard), lambda t: (0, 0)),
            pl.BlockSpec((k_shard, nt), lambda t: (0, t)),
            pl.BlockSpec((1, 1), lambda t: (0, 0), memory_space=pltpu.SMEM),
            pl.BlockSpec((1, 1), lambda t: (0, 0), memory_space=pltpu.SMEM),
        ],
        out_specs=pl.BlockSpec((m_per, nt), lambda t: (0, t)),
        out_shape=jax.ShapeDtypeStruct((m_per, n), jnp.float32),
        scratch_shapes=[
            pltpu.VMEM((k_shard, nt), jnp.bfloat16),
            pltpu.VMEM((m_per, nt), jnp.bfloat16),
            pltpu.VMEM((N_DEV - 1, m_per, nt), jnp.bfloat16),
            pltpu.SemaphoreType.DMA((N_DEV - 1,)),
            pltpu.SemaphoreType.DMA((N_DEV - 1,)),
        ],
        compiler_params=pltpu.CompilerParams(
            collective_id=0,
            dimension_semantics=("arbitrary",),
        ),
    )(x, w_mat, scale_x.reshape(1, 1), scale_w.reshape(1, 1))


# baseline (device time: 669152 ns/iter reference)
import jax
import jax.numpy as jnp
from jax import lax
from jax.experimental import pallas as pl
from jax.experimental.pallas import tpu as pltpu

N_DEV = 4
N_TILES = 4


def kernel(x, w_mat, scale_x, scale_w):
    m_global, k_shard = x.shape
    _, n = w_mat.shape
    m_per = m_global // N_DEV
    nt = n // N_TILES

    def body(x_ref, w_ref, sx_ref, sw_ref, out_ref,
             w_bf, send_buf, recv_buf, send_sems, recv_sems):
        t = pl.program_id(0)
        d = lax.axis_index("i")
        left = lax.rem(d + N_DEV - 1, N_DEV)
        right = lax.rem(d + 1, N_DEV)

        @pl.when(t == 0)
        def _():
            barrier_sem = pltpu.get_barrier_semaphore()
            pl.semaphore_signal(barrier_sem, inc=1, device_id=(left,),
                                device_id_type=pl.DeviceIdType.MESH)
            pl.semaphore_signal(barrier_sem, inc=1, device_id=(right,),
                                device_id_type=pl.DeviceIdType.MESH)
            pl.semaphore_wait(barrier_sem, 2)

        w_bf[...] = w_ref[...].astype(jnp.bfloat16)

        def partial(c):
            xc = x_ref[pl.ds(c * m_per, m_per), :].astype(jnp.bfloat16)
            return jnp.dot(xc, w_bf[...], preferred_element_type=jnp.float32)

        for s in range(N_DEV - 1):
            c = lax.rem(d + 2 * N_DEV - 1 - s, N_DEV)
            acc = partial(c)
            if s > 0:
                acc = acc + recv_buf[s - 1].astype(jnp.float32)
            send_buf[...] = acc.astype(jnp.bfloat16)
            rdma = pltpu.make_async_remote_copy(
                src_ref=send_buf,
                dst_ref=recv_buf.at[s],
                send_sem=send_sems.at[s],
                recv_sem=recv_sems.at[s],
                device_id=(right,),
                device_id_type=pl.DeviceIdType.MESH,
            )
            rdma.start()
            rdma.wait()

        acc = partial(d) + recv_buf[N_DEV - 2].astype(jnp.float32)
        scale = sx_ref[0, 0] * sw_ref[0, 0]
        out_ref[...] = jnp.maximum(acc * scale, 0.0)

    return pl.pallas_call(
        body,
        grid=(N_TILES,),
        in_specs=[
            pl.BlockSpec((m_global, k_shard), lambda t: (0, 0)),
            pl.BlockSpec((k_shard, nt), lambda t: (0, t)),
            pl.BlockSpec((1, 1), lambda t: (0, 0), memory_space=pltpu.SMEM),
            pl.BlockSpec((1, 1), lambda t: (0, 0), memory_space=pltpu.SMEM),
        ],
        out_specs=pl.BlockSpec((m_per, nt), lambda t: (0, t)),
        out_shape=jax.ShapeDtypeStruct((m_per, n), jnp.float32),
        scratch_shapes=[
            pltpu.VMEM((k_shard, nt), jnp.bfloat16),
            pltpu.VMEM((m_per, nt), jnp.bfloat16),
            pltpu.VMEM((N_DEV - 1, m_per, nt), jnp.bfloat16),
            pltpu.SemaphoreType.DMA((N_DEV - 1,)),
            pltpu.SemaphoreType.DMA((N_DEV - 1,)),
        ],
        compiler_params=pltpu.CompilerParams(
            collective_id=0,
            dimension_semantics=("arbitrary",),
            vmem_limit_bytes=56 * 1024 * 1024,
        ),
    )(x, w_mat, scale_x.reshape(1, 1), scale_w.reshape(1, 1))


# device time: 598098 ns/iter; 1.1188x vs baseline; 1.1188x over previous
import jax
import jax.numpy as jnp
from jax import lax
from jax.experimental import pallas as pl
from jax.experimental.pallas import tpu as pltpu

N_DEV = 4
N_TILES = 4
K_HALF = 2


def kernel(x, w_mat, scale_x, scale_w):
    m_global, k_shard = x.shape
    _, n = w_mat.shape
    m_per = m_global // N_DEV
    nt = n // N_TILES
    nh = nt // K_HALF

    def body(x_ref, w_ref, sx_ref, sw_ref, out_ref,
             w_bf, send_buf, recv_buf, own_bf, stage,
             send_sems, recv_sems, copy_sem):
        d = lax.axis_index("i")
        left = lax.rem(d + N_DEV - 1, N_DEV)
        right = lax.rem(d + 1, N_DEV)

        barrier_sem = pltpu.get_barrier_semaphore()
        pl.semaphore_signal(barrier_sem, inc=1, device_id=(left,),
                            device_id_type=pl.DeviceIdType.MESH)
        pl.semaphore_signal(barrier_sem, inc=1, device_id=(right,),
                            device_id_type=pl.DeviceIdType.MESH)
        pl.semaphore_wait(barrier_sem, 2)

        def load_w_tile(t):
            for h in range(K_HALF):
                w_bf[:, pl.ds(h * nh, nh)] = w_ref[
                    :, pl.ds(t * nt + h * nh, nh)].astype(jnp.bfloat16)

        def partial_to(dst, c):
            xc = x_ref[pl.ds(c * m_per, m_per), :].astype(jnp.bfloat16)
            for h in range(K_HALF):
                ph = jnp.dot(xc, w_bf[:, pl.ds(h * nh, nh)],
                             preferred_element_type=jnp.float32)
                dst[:, pl.ds(h * nh, nh)] = ph.astype(jnp.bfloat16)

        def send_rdma(j):
            return pltpu.make_async_remote_copy(
                src_ref=send_buf.at[j % 2],
                dst_ref=recv_buf.at[j % 2],
                send_sem=send_sems.at[j % 2],
                recv_sem=recv_sems.at[j % 2],
                device_id=(right,),
                device_id_type=pl.DeviceIdType.MESH,
            )

        rdmas = {}
        copies = {}
        c0 = lax.rem(d + N_DEV - 1, N_DEV)

        def issue(j, c):
            if j >= 2:
                rdmas[j - 2].wait_send()
            partial_to(send_buf.at[j % 2], c)
            r = send_rdma(j)
            r.start()
            rdmas[j] = r

        load_w_tile(0)
        issue(0, c0)

        for t in range(N_TILES):
            for s in (1, 2):
                j = 3 * t + s
                c = lax.rem(d + 2 * N_DEV - 1 - s, N_DEV)
                if j >= 2:
                    rdmas[j - 2].wait_send()
                partial_to(send_buf.at[j % 2], c)
                rdmas[j - 1].wait_recv()
                send_buf[j % 2] = send_buf[j % 2] + recv_buf[(j - 1) % 2]
                r = send_rdma(j)
                r.start()
                rdmas[j] = r

            partial_to(own_bf, d)

            if t < N_TILES - 1:
                load_w_tile(t + 1)
                issue(3 * (t + 1), c0)

            jf = 3 * t + 2
            rdmas[jf].wait_recv()
            scale = sx_ref[0, 0] * sw_ref[0, 0]
            acc = own_bf[...] + recv_buf[jf % 2]
            if t > 0:
                copies[t - 1].wait()
            stage[...] = jnp.maximum(acc.astype(jnp.float32) * scale, 0.0)
            cp = pltpu.make_async_copy(
                stage, out_ref.at[:, pl.ds(t * nt, nt)], copy_sem)
            cp.start()
            copies[t] = cp

        rdmas[3 * N_TILES - 2].wait_send()
        rdmas[3 * N_TILES - 1].wait_send()
        copies[N_TILES - 1].wait()

    return pl.pallas_call(
        body,
        in_specs=[
            pl.BlockSpec(memory_space=pltpu.VMEM),
            pl.BlockSpec(memory_space=pltpu.VMEM),
            pl.BlockSpec(memory_space=pltpu.SMEM),
            pl.BlockSpec(memory_space=pltpu.SMEM),
        ],
        out_specs=pl.BlockSpec(memory_space=pl.ANY),
        out_shape=jax.ShapeDtypeStruct((m_per, n), jnp.float32),
        scratch_shapes=[
            pltpu.VMEM((k_shard, nt), jnp.bfloat16),
            pltpu.VMEM((2, m_per, nt), jnp.bfloat16),
            pltpu.VMEM((2, m_per, nt), jnp.bfloat16),
            pltpu.VMEM((m_per, nt), jnp.bfloat16),
            pltpu.VMEM((m_per, nt), jnp.float32),
            pltpu.SemaphoreType.DMA((2,)),
            pltpu.SemaphoreType.DMA((2,)),
            pltpu.SemaphoreType.DMA,
        ],
        compiler_params=pltpu.CompilerParams(
            collective_id=0,
            vmem_limit_bytes=52 * 1024 * 1024,
        ),
    )(x, w_mat, scale_x.reshape(1, 1), scale_w.reshape(1, 1))


# device time: 578040 ns/iter; 1.1576x vs baseline; 1.0347x over previous
import jax
import jax.numpy as jnp
from jax import lax
from jax.experimental import pallas as pl
from jax.experimental.pallas import tpu as pltpu

N_DEV = 4
N_TILES = 4
N_HALF = 2


def kernel(x, w_mat, scale_x, scale_w):
    m_global, k_shard = x.shape
    _, n = w_mat.shape
    m_per = m_global // N_DEV
    nt = n // N_TILES
    nh = nt // N_HALF

    def body(x_ref, w_ref, sx_ref, sw_ref, out_ref,
             w_bf, send_buf, recv_buf, own_bf, stage,
             send_sems, recv_sems, copy_sem):
        d = lax.axis_index("i")
        left = lax.rem(d + N_DEV - 1, N_DEV)
        right = lax.rem(d + 1, N_DEV)

        barrier_sem = pltpu.get_barrier_semaphore()
        pl.semaphore_signal(barrier_sem, inc=1, device_id=(left,),
                            device_id_type=pl.DeviceIdType.MESH)
        pl.semaphore_signal(barrier_sem, inc=1, device_id=(right,),
                            device_id_type=pl.DeviceIdType.MESH)
        pl.semaphore_wait(barrier_sem, 2)

        def load_w_tile(t):
            for h in range(N_HALF):
                w_bf[:, pl.ds(h * nh, nh)] = w_ref[
                    :, pl.ds(t * nt + h * nh, nh)].astype(jnp.bfloat16)

        def partial_half(c, h):
            xc = x_ref[pl.ds(c * m_per, m_per), :].astype(jnp.bfloat16)
            ph = jnp.dot(xc, w_bf[:, pl.ds(h * nh, nh)],
                         preferred_element_type=jnp.float32)
            return ph.astype(jnp.bfloat16)

        def half_rdma(j, h):
            return pltpu.make_async_remote_copy(
                src_ref=send_buf.at[j % 2, :, pl.ds(h * nh, nh)],
                dst_ref=recv_buf.at[j % 2, :, pl.ds(h * nh, nh)],
                send_sem=send_sems.at[j % 2, h],
                recv_sem=recv_sems.at[j % 2, h],
                device_id=(right,),
                device_id_type=pl.DeviceIdType.MESH,
            )

        rdmas = {}
        copies = {}
        c0 = lax.rem(d + N_DEV - 1, N_DEV)

        def issue_hop0(j, c):
            for h in range(N_HALF):
                if j >= 2:
                    rdmas[j - 2, h].wait_send()
                send_buf[j % 2, :, pl.ds(h * nh, nh)] = partial_half(c, h)
                r = half_rdma(j, h)
                r.start()
                rdmas[j, h] = r

        load_w_tile(0)
        issue_hop0(0, c0)

        for t in range(N_TILES):
            for s in (1, 2):
                j = 3 * t + s
                c = lax.rem(d + 2 * N_DEV - 1 - s, N_DEV)
                for h in range(N_HALF):
                    if j >= 2:
                        rdmas[j - 2, h].wait_send()
                    send_buf[j % 2, :, pl.ds(h * nh, nh)] = partial_half(c, h)
                for h in range(N_HALF):
                    hs = pl.ds(h * nh, nh)
                    rdmas[j - 1, h].wait_recv()
                    send_buf[j % 2, :, hs] = (
                        send_buf[j % 2, :, hs] + recv_buf[(j - 1) % 2, :, hs])
                    r = half_rdma(j, h)
                    r.start()
                    rdmas[j, h] = r

            for h in range(N_HALF):
                own_bf[:, pl.ds(h * nh, nh)] = partial_half(d, h)

            if t < N_TILES - 1:
                load_w_tile(t + 1)
                issue_hop0(3 * (t + 1), c0)

            jf = 3 * t + 2
            scale = sx_ref[0, 0] * sw_ref[0, 0]
            if t > 0:
                copies[t - 1].wait()
            for h in range(N_HALF):
                hs = pl.ds(h * nh, nh)
                rdmas[jf, h].wait_recv()
                acc = own_bf[:, hs] + recv_buf[jf % 2, :, hs]
                stage[:, hs] = jnp.maximum(
                    acc.astype(jnp.float32) * scale, 0.0)
            cp = pltpu.make_async_copy(
                stage, out_ref.at[:, pl.ds(t * nt, nt)], copy_sem)
            cp.start()
            copies[t] = cp

        for j in (3 * N_TILES - 2, 3 * N_TILES - 1):
            for h in range(N_HALF):
                rdmas[j, h].wait_send()
        copies[N_TILES - 1].wait()

    return pl.pallas_call(
        body,
        in_specs=[
            pl.BlockSpec(memory_space=pltpu.VMEM),
            pl.BlockSpec(memory_space=pltpu.VMEM),
            pl.BlockSpec(memory_space=pltpu.SMEM),
            pl.BlockSpec(memory_space=pltpu.SMEM),
        ],
        out_specs=pl.BlockSpec(memory_space=pl.ANY),
        out_shape=jax.ShapeDtypeStruct((m_per, n), jnp.float32),
        scratch_shapes=[
            pltpu.VMEM((k_shard, nt), jnp.bfloat16),
            pltpu.VMEM((2, m_per, nt), jnp.bfloat16),
            pltpu.VMEM((2, m_per, nt), jnp.bfloat16),
            pltpu.VMEM((m_per, nt), jnp.bfloat16),
            pltpu.VMEM((m_per, nt), jnp.float32),
            pltpu.SemaphoreType.DMA((2, N_HALF)),
            pltpu.SemaphoreType.DMA((2, N_HALF)),
            pltpu.SemaphoreType.DMA,
        ],
        compiler_params=pltpu.CompilerParams(
            collective_id=0,
            vmem_limit_bytes=52 * 1024 * 1024,
        ),
    )(x, w_mat, scale_x.reshape(1, 1), scale_w.reshape(1, 1))


# device time: 306927 ns/iter; 2.1802x vs baseline; 1.8833x over previous
import jax
import jax.numpy as jnp
from jax import lax
from jax.experimental import pallas as pl
from jax.experimental.pallas import tpu as pltpu

N_DEV = 4
N_DIR = 2
TILES_PER_DIR = 4
N_HALF = 2


def kernel(x, w_mat, scale_x, scale_w):
    m_global, k_shard = x.shape
    _, n = w_mat.shape
    m_per = m_global // N_DEV
    ncols_dir = n // N_DIR
    nt = ncols_dir // TILES_PER_DIR
    nh = nt // N_HALF

    def body(x_ref, w_ref, sx_ref, sw_ref, out_ref,
             w_bf, send_buf, recv_buf, own_bf, stage,
             send_sems, recv_sems, copy_sems):
        d = lax.axis_index("i")
        left = lax.rem(d + N_DEV - 1, N_DEV)
        right = lax.rem(d + 1, N_DEV)

        barrier_sem = pltpu.get_barrier_semaphore()
        pl.semaphore_signal(barrier_sem, inc=1, device_id=(left,),
                            device_id_type=pl.DeviceIdType.MESH)
        pl.semaphore_signal(barrier_sem, inc=1, device_id=(right,),
                            device_id_type=pl.DeviceIdType.MESH)
        pl.semaphore_wait(barrier_sem, 2)

        target = {0: right, 1: left}

        def chunk_at_hop(dir_, s):
            if dir_ == 0:
                return lax.rem(d + 2 * N_DEV - 1 - s, N_DEV)
            return lax.rem(d + 1 + s, N_DEV)

        def col0(dir_, t):
            return dir_ * ncols_dir + t * nt

        def load_w_tile(dir_, t):
            w_bf[dir_] = w_ref[:, pl.ds(col0(dir_, t), nt)].astype(
                jnp.bfloat16)

        def partial_half(dir_, c, h):
            xc = x_ref[pl.ds(c * m_per, m_per), :].astype(jnp.bfloat16)
            ph = jnp.dot(xc, w_bf[dir_, :, pl.ds(h * nh, nh)],
                         preferred_element_type=jnp.float32)
            return ph.astype(jnp.bfloat16)

        def half_rdma(dir_, j, h):
            return pltpu.make_async_remote_copy(
                src_ref=send_buf.at[dir_, j % 2, :, pl.ds(h * nh, nh)],
                dst_ref=recv_buf.at[dir_, j % 2, :, pl.ds(h * nh, nh)],
                send_sem=send_sems.at[dir_, j % 2, h],
                recv_sem=recv_sems.at[dir_, j % 2, h],
                device_id=(target[dir_],),
                device_id_type=pl.DeviceIdType.MESH,
            )

        rdmas = {}
        copies = {}

        def issue_hop0(dir_, j):
            c = chunk_at_hop(dir_, 0)
            for h in range(N_HALF):
                if j >= 2:
                    rdmas[dir_, j - 2, h].wait_send()
                send_buf[dir_, j % 2, :, pl.ds(h * nh, nh)] = \
                    partial_half(dir_, c, h)
                r = half_rdma(dir_, j, h)
                r.start()
                rdmas[dir_, j, h] = r

        for dir_ in range(N_DIR):
            load_w_tile(dir_, 0)
            issue_hop0(dir_, 0)

        for t in range(TILES_PER_DIR):
            for s in (1, 2):
                j = 3 * t + s
                for dir_ in range(N_DIR):
                    c = chunk_at_hop(dir_, s)
                    for h in range(N_HALF):
                        if j >= 2:
                            rdmas[dir_, j - 2, h].wait_send()
                        send_buf[dir_, j % 2, :, pl.ds(h * nh, nh)] = \
                            partial_half(dir_, c, h)
                for h in range(N_HALF):
                    for dir_ in range(N_DIR):
                        hs = pl.ds(h * nh, nh)
                        rdmas[dir_, j - 1, h].wait_recv()
                        send_buf[dir_, j % 2, :, hs] = (
                            send_buf[dir_, j % 2, :, hs]
                            + recv_buf[dir_, (j - 1) % 2, :, hs])
                        r = half_rdma(dir_, j, h)
                        r.start()
                        rdmas[dir_, j, h] = r

            for dir_ in range(N_DIR):
                for h in range(N_HALF):
                    own_bf[dir_, :, pl.ds(h * nh, nh)] = \
                        partial_half(dir_, d, h)

            if t < TILES_PER_DIR - 1:
                for dir_ in range(N_DIR):
                    load_w_tile(dir_, t + 1)
                    issue_hop0(dir_, 3 * (t + 1))

            jf = 3 * t + 2
            scale = sx_ref[0, 0] * sw_ref[0, 0]
            for dir_ in range(N_DIR):
                if t > 0:
                    copies[dir_, t - 1].wait()
                for h in range(N_HALF):
                    hs = pl.ds(h * nh, nh)
                    rdmas[dir_, jf, h].wait_recv()
                    acc = own_bf[dir_, :, hs] + recv_buf[dir_, jf % 2, :, hs]
                    stage[dir_, :, hs] = jnp.maximum(
                        acc.astype(jnp.float32) * scale, 0.0)
                cp = pltpu.make_async_copy(
                    stage.at[dir_],
                    out_ref.at[:, pl.ds(col0(dir_, t), nt)],
                    copy_sems.at[dir_])
                cp.start()
                copies[dir_, t] = cp

        for dir_ in range(N_DIR):
            for j in (3 * TILES_PER_DIR - 2, 3 * TILES_PER_DIR - 1):
                for h in range(N_HALF):
                    rdmas[dir_, j, h].wait_send()
            copies[dir_, TILES_PER_DIR - 1].wait()

    return pl.pallas_call(
        body,
        in_specs=[
            pl.BlockSpec(memory_space=pltpu.VMEM),
            pl.BlockSpec(memory_space=pltpu.VMEM),
            pl.BlockSpec(memory_space=pltpu.SMEM),
            pl.BlockSpec(memory_space=pltpu.SMEM),
        ],
        out_specs=pl.BlockSpec(memory_space=pl.ANY),
        out_shape=jax.ShapeDtypeStruct((m_per, n), jnp.float32),
        scratch_shapes=[
            pltpu.VMEM((N_DIR, k_shard, nt), jnp.bfloat16),
            pltpu.VMEM((N_DIR, 2, m_per, nt), jnp.bfloat16),
            pltpu.VMEM((N_DIR, 2, m_per, nt), jnp.bfloat16),
            pltpu.VMEM((N_DIR, m_per, nt), jnp.bfloat16),
            pltpu.VMEM((N_DIR, m_per, nt), jnp.float32),
            pltpu.SemaphoreType.DMA((N_DIR, 2, N_HALF)),
            pltpu.SemaphoreType.DMA((N_DIR, 2, N_HALF)),
            pltpu.SemaphoreType.DMA((N_DIR,)),
        ],
        compiler_params=pltpu.CompilerParams(
            collective_id=0,
            vmem_limit_bytes=52 * 1024 * 1024,
        ),
    )(x, w_mat, scale_x.reshape(1, 1), scale_w.reshape(1, 1))
